# baseline (speedup 1.0000x reference)
import functools

import jax
import jax.numpy as jnp
from jax import lax
from jax.experimental import pallas as pl
from jax.experimental.pallas import tpu as pltpu
from jax.experimental.pallas import tpu_sc as plsc

_B = 16384
_D = 32
_NC = 2
_NS = 16
_NW = _NC * _NS
_BPW = _B // _NW
_NG = _BPW // 16
_DCH = 4
_NSL = _D // _DCH

_mesh = plsc.VectorSubcoreMesh(core_axis_name="c", subcore_axis_name="s")
_params = pltpu.CompilerParams(
    use_tc_tiling_on_sc=False, needs_layout_passes=False)


def _strategy_half(sid, semb):
    @functools.partial(
        pl.kernel,
        mesh=_mesh,
        out_type=jax.ShapeDtypeStruct((_D, _B), jnp.float32),
        compiler_params=_params,
        scratch_types=[
            pltpu.VMEM((_BPW,), jnp.int32),
            pltpu.VMEM((8, _D), jnp.float32),
            pltpu.VMEM((_D, _BPW), jnp.float32),
        ],
    )
    def body(sid_hbm, semb_hbm, out_hbm, sidx, sv, combt):
        wid = lax.axis_index("s") * _NC + lax.axis_index("c")
        base = wid * _BPW
        pltpu.sync_copy(sid_hbm.at[wid], sidx)
        pltpu.sync_copy(semb_hbm, sv)
        cols = [jnp.full((16,), c, jnp.int32) for c in range(_D)]

        def sgroup(g, _):
            sid16 = sidx[pl.ds(g * 16, 16)]
            for d in range(_D):
                combt[d, pl.ds(g * 16, 16)] = plsc.load_gather(
                    sv, [sid16, cols[d]])
            return _

        lax.fori_loop(0, _NG, sgroup, None)
        pltpu.sync_copy(combt, out_hbm.at[:, pl.ds(base, _BPW)])

    return body(sid, semb)


def _cause_half(cid, cembt):
    @functools.partial(
        pl.kernel,
        mesh=_mesh,
        out_type=jax.ShapeDtypeStruct((_D, _B), jnp.float32),
        compiler_params=_params,
        scratch_types=[
            pltpu.VMEM((_BPW,), jnp.int32),
            pltpu.VMEM((_D, _BPW), jnp.float32),
            pltpu.VMEM_SHARED((_DCH, 100000), jnp.float32),
            pltpu.VMEM_SHARED((_DCH, 100000), jnp.float32),
            pltpu.SemaphoreType.DMA,
            pltpu.SemaphoreType.DMA,
        ],
    )
    def body(cid_hbm, cembt_hbm, out_hbm, cidx, combt, slab0, slab1,
             gsem, ssem):
        sub = lax.axis_index("s")
        wid = sub * _NC + lax.axis_index("c")
        base = wid * _BPW
        pltpu.sync_copy(cid_hbm.at[wid], cidx)

        slabs = [slab0, slab1]

        @pl.when(sub < _DCH)
        def _():
            pltpu.async_copy(cembt_hbm.at[sub], slab0.at[sub], ssem)

        for k in range(_NSL):
            cur = slabs[k % 2]

            @pl.when(sub < _DCH)
            def _():
                pltpu.make_async_copy(
                    cembt_hbm.at[k * _DCH + sub], cur.at[sub], ssem).wait()

            plsc.subcore_barrier()

            if k + 1 < _NSL:
                nxt = slabs[(k + 1) % 2]

                @pl.when(sub < _DCH)
                def _():
                    pltpu.async_copy(
                        cembt_hbm.at[(k + 1) * _DCH + sub],
                        nxt.at[sub], ssem)

            copies = []
            for d in range(_DCH):
                copies.append(pltpu.async_copy(
                    cur.at[d].at[cidx],
                    combt.at[k * _DCH + d], gsem))
            for c in copies:
                c.wait()
            plsc.subcore_barrier()

        pltpu.sync_copy(combt, out_hbm.at[:, pl.ds(base, _BPW)])

    return body(cid, cembt)


def kernel(strategy_id, cause_index, strategy_emb, cause_emb):
    sid = strategy_id.astype(jnp.int32).reshape(_NW, _BPW)
    cid = cause_index.astype(jnp.int32).reshape(_NW, _BPW)
    st = _strategy_half(sid, strategy_emb)
    ct = _cause_half(cid, cause_emb.T)
    return jnp.concatenate([st, ct], axis=0).T


# speedup vs baseline: 1.1593x; 1.1593x over previous
import functools

import jax
import jax.numpy as jnp
from jax import lax
from jax.experimental import pallas as pl
from jax.experimental.pallas import tpu as pltpu
from jax.experimental.pallas import tpu_sc as plsc

_B = 16384
_D = 32
_NC = 2
_NS = 16
_NW = _NC * _NS
_BPW = _B // _NW
_NG = _BPW // 16
_DCH = 4
_NSL = _D // _DCH

_mesh = plsc.VectorSubcoreMesh(core_axis_name="c", subcore_axis_name="s")
_params = pltpu.CompilerParams(
    use_tc_tiling_on_sc=False, needs_layout_passes=False)


def _strategy_half(sid, semb):
    @functools.partial(
        pl.kernel,
        mesh=_mesh,
        out_type=jax.ShapeDtypeStruct((_D, _B), jnp.float32),
        compiler_params=_params,
        scratch_types=[
            pltpu.VMEM((_BPW,), jnp.int32),
            pltpu.VMEM((8, _D), jnp.float32),
            pltpu.VMEM((8 * _D,), jnp.float32),
            pltpu.VMEM((_D, _BPW), jnp.float32),
        ],
    )
    def body(sid_hbm, semb_hbm, out_hbm, sidx, sv2, sv, combt):
        wid = lax.axis_index("s") * _NC + lax.axis_index("c")
        base = wid * _BPW
        pltpu.sync_copy(sid_hbm.at[wid], sidx)
        pltpu.sync_copy(semb_hbm, sv2)
        for i in range(8):
            for j in range(_D // 16):
                sv[pl.ds(i * _D + j * 16, 16)] = sv2[i, pl.ds(j * 16, 16)]

        def sgroup(g, _):
            sofs = sidx[pl.ds(g * 16, 16)] * _D
            for d in range(_D):
                combt[d, pl.ds(g * 16, 16)] = plsc.load_gather(
                    sv, [sofs + d])
            return _

        lax.fori_loop(0, _NG, sgroup, None)
        pltpu.sync_copy(combt, out_hbm.at[:, pl.ds(base, _BPW)])

    return body(sid, semb)


def _cause_half(cid, cembt, st):
    @functools.partial(
        pl.kernel,
        mesh=_mesh,
        out_type=jax.ShapeDtypeStruct((2 * _D, _B), jnp.float32),
        compiler_params=_params,
        scratch_types=[
            pltpu.VMEM((_BPW,), jnp.int32),
            pltpu.VMEM((2 * _D, _BPW), jnp.float32),
            pltpu.VMEM_SHARED((_DCH, 100000), jnp.float32),
            pltpu.VMEM_SHARED((_DCH, 100000), jnp.float32),
            pltpu.SemaphoreType.DMA,
            pltpu.SemaphoreType.DMA,
        ],
    )
    def body(cid_hbm, cembt_hbm, st_hbm, out_hbm, cidx, combt,
             slab0, slab1, gsem, ssem):
        sub = lax.axis_index("s")
        wid = sub * _NC + lax.axis_index("c")
        base = wid * _BPW
        pltpu.sync_copy(cid_hbm.at[wid], cidx)
        pltpu.sync_copy(st_hbm.at[:, pl.ds(base, _BPW)],
                        combt.at[pl.ds(0, _D)])

        slabs = [slab0, slab1]

        @pl.when(sub < _DCH)
        def _():
            pltpu.async_copy(cembt_hbm.at[sub], slab0.at[sub], ssem)

        for k in range(_NSL):
            cur = slabs[k % 2]

            @pl.when(sub < _DCH)
            def _():
                pltpu.make_async_copy(
                    cembt_hbm.at[k * _DCH + sub], cur.at[sub], ssem).wait()

            plsc.subcore_barrier()

            if k + 1 < _NSL:
                nxt = slabs[(k + 1) % 2]

                @pl.when(sub < _DCH)
                def _():
                    pltpu.async_copy(
                        cembt_hbm.at[(k + 1) * _DCH + sub],
                        nxt.at[sub], ssem)

            copies = []
            for d in range(_DCH):
                copies.append(pltpu.async_copy(
                    cur.at[d].at[cidx],
                    combt.at[_D + k * _DCH + d], gsem))
            for c in copies:
                c.wait()
            plsc.subcore_barrier()

        pltpu.sync_copy(combt, out_hbm.at[:, pl.ds(base, _BPW)])

    return body(cid, cembt, st)


def kernel(strategy_id, cause_index, strategy_emb, cause_emb):
    sid = strategy_id.astype(jnp.int32).reshape(_NW, _BPW)
    cid = cause_index.astype(jnp.int32).reshape(_NW, _BPW)
    st = _strategy_half(sid, strategy_emb)
    return _cause_half(cid, cause_emb.T, st).T
